# Initial kernel scaffold; baseline (speedup 1.0000x reference)
import jax
import jax.numpy as jnp
from jax.experimental import pallas as pl


def kernel(x, layer_idx, requested_r):
    raise NotImplementedError("write your pallas kernel here")


# speedup vs baseline: 3.7349x; 3.7349x over previous
import functools

import jax
import jax.numpy as jnp
from jax import lax
from jax.experimental import pallas as pl
from jax.experimental.pallas import tpu as pltpu
from jax.experimental.pallas import tpu_sc as plsc

_B, _T, _C = 32, 1025, 768
_K = _T - 256
_PPAD = 784
_CHUNK = 64
_SPAN = 776
_NFULL = _SPAN // _CHUNK
_TAIL = _SPAN - _NFULL * _CHUNK


def _topk_body(x_ref, idx_ref):
    xb = x_ref[0]
    rs = []
    for c in range(_C // 128):
        ch = xb[:, c * 128:(c + 1) * 128]
        rs.append(jnp.sum(ch * ch, axis=-1))
    q = rs[0]
    for c in range(1, _C // 128):
        q = q + rs[c]
    s = jnp.sqrt(q)

    ii1 = lax.broadcasted_iota(jnp.int32, (_T, 1), 0)
    jj1 = lax.broadcasted_iota(jnp.int32, (1, _T), 1)
    sc = jnp.where(ii1 == 0, jnp.inf, s[:, None])
    sr = jnp.where(jj1 == 0, jnp.inf, s[None, :])

    ii = lax.broadcasted_iota(jnp.int32, (_T, _T), 0)
    jj = lax.broadcasted_iota(jnp.int32, (_T, _T), 1)
    beats = jnp.where((sr > sc) | ((sr == sc) & (jj < ii)), 1.0, 0.0)
    rank = jnp.sum(beats, axis=1, keepdims=True)
    rank32 = rank.astype(jnp.int32)

    pr = lax.broadcasted_iota(jnp.int32, (_T, _PPAD), 1)
    it = lax.broadcasted_iota(jnp.int32, (_T, _PPAD), 0)
    loc = jnp.sum(jnp.where(rank32 == pr, it, 0), axis=0)
    idx_ref[0, 0, :] = loc + pl.program_id(0) * _T


_topk_call = pl.pallas_call(
    _topk_body,
    grid=(_B,),
    in_specs=[pl.BlockSpec((1, _T, _C), lambda b: (b, 0, 0))],
    out_specs=pl.BlockSpec((1, 1, _PPAD), lambda b: (b, 0, 0)),
    out_shape=jax.ShapeDtypeStruct((_B, 1, _PPAD), jnp.int32),
)


def _gather_body(xflat, idxf, out, idx_v, buf0, buf1, tail_v, g0, g1, s0, s1):
    wid = lax.axis_index("s") * 2 + lax.axis_index("c")
    base = pl.multiple_of((wid * _K >> 3) << 3, 8)
    pltpu.sync_copy(idxf.at[pl.ds(base, _SPAN)], idx_v)
    bufs = (buf0, buf1)
    gsem = (g0, g1)
    ssem = (s0, s1)

    def start_gather(c):
        if c == _NFULL:
            return pltpu.async_copy(xflat.at[idx_v.at[pl.ds(_NFULL * _CHUNK, _TAIL)]],
                                    tail_v, gsem[c % 2])
        return pltpu.async_copy(xflat.at[idx_v.at[pl.ds(c * _CHUNK, _CHUNK)]],
                                bufs[c % 2], gsem[c % 2])

    def start_store(c):
        if c == _NFULL:
            return pltpu.async_copy(tail_v, out.at[pl.ds(base + _NFULL * _CHUNK, _TAIL)],
                                    ssem[c % 2])
        return pltpu.async_copy(bufs[c % 2], out.at[pl.ds(base + c * _CHUNK, _CHUNK)],
                                ssem[c % 2])

    nch = _NFULL + 1
    g_h = [None] * nch
    s_h = [None] * nch
    g_h[0] = start_gather(0)
    for c in range(nch):
        if c + 1 < nch:
            if c - 1 >= 0 and c + 1 < _NFULL:
                s_h[c - 1].wait()
            g_h[c + 1] = start_gather(c + 1)
        g_h[c].wait()
        s_h[c] = start_store(c)
    s_h[nch - 3].wait()
    s_h[nch - 2].wait()
    s_h[nch - 1].wait()


@functools.lru_cache(maxsize=1)
def _make_gather_call():
    return functools.partial(
        pl.kernel,
        out_type=jax.ShapeDtypeStruct((_B * _K, _C), jnp.float32),
        mesh=plsc.VectorSubcoreMesh(core_axis_name="c", subcore_axis_name="s"),
        scratch_types=[
            pltpu.VMEM((_SPAN,), jnp.int32),
            pltpu.VMEM((_CHUNK, _C), jnp.float32),
            pltpu.VMEM((_CHUNK, _C), jnp.float32),
            pltpu.VMEM((_TAIL, _C), jnp.float32),
            pltpu.SemaphoreType.DMA,
            pltpu.SemaphoreType.DMA,
            pltpu.SemaphoreType.DMA,
            pltpu.SemaphoreType.DMA,
        ],
    )(_gather_body)


def kernel(x, layer_idx, requested_r):
    del layer_idx
    idx3 = _topk_call(x)
    idx2 = idx3.reshape(_B, _PPAD)[:, :_K] + (requested_r - 256)
    idxf = idx2.astype(jnp.int32).reshape(_B * _K)
    xflat = x.reshape(_B * _T, _C)
    outflat = _make_gather_call()(xflat, idxf)
    return outflat.reshape(_B, _K, _C)


# speedup vs baseline: 5.9374x; 1.5897x over previous
import functools

import jax
import jax.numpy as jnp
from jax import lax
from jax.experimental import pallas as pl
from jax.experimental.pallas import tpu as pltpu
from jax.experimental.pallas import tpu_sc as plsc

_B, _T, _C = 32, 1025, 768
_K = _T - 256
_PPAD = 784
_CHUNK = 64
_NFULL = _K // _CHUNK


def _topk_body(r_ref, x_ref, idx_ref):
    xb = x_ref[0]
    rs = []
    for c in range(_C // 128):
        ch = xb[:, c * 128:(c + 1) * 128]
        rs.append(jnp.sum(ch * ch, axis=-1))
    q = rs[0]
    for c in range(1, _C // 128):
        q = q + rs[c]
    s = jnp.sqrt(q)

    ii1 = lax.broadcasted_iota(jnp.int32, (_T, 1), 0)
    jj1 = lax.broadcasted_iota(jnp.int32, (1, _T), 1)
    sc = jnp.where(ii1 == 0, jnp.inf, s[:, None])
    sr = jnp.where(jj1 == 0, jnp.inf, s[None, :])

    ii = lax.broadcasted_iota(jnp.int32, (_T, _T), 0)
    jj = lax.broadcasted_iota(jnp.int32, (_T, _T), 1)
    beats = jnp.where((sr > sc) | ((sr == sc) & (jj < ii)), 1.0, 0.0)
    rank = jnp.sum(beats, axis=1, keepdims=True)
    rank32 = rank.astype(jnp.int32)

    pr = lax.broadcasted_iota(jnp.int32, (_T, _PPAD), 1)
    it = lax.broadcasted_iota(jnp.int32, (_T, _PPAD), 0)
    loc = jnp.sum(jnp.where(rank32 == pr, it, 0), axis=0)
    idx_ref[0, 0, :] = loc + (r_ref[0] - 256)


_topk_call = pl.pallas_call(
    _topk_body,
    grid=(_B,),
    in_specs=[
        pl.BlockSpec(memory_space=pltpu.SMEM),
        pl.BlockSpec((1, _T, _C), lambda b: (b, 0, 0)),
    ],
    out_specs=pl.BlockSpec((1, 1, _PPAD), lambda b: (b, 0, 0)),
    out_shape=jax.ShapeDtypeStruct((_B, 1, _PPAD), jnp.int32),
)


def _gather_body(x, idxp, out, idx_v, buf0, buf1, tail_v, g0, g1, s0, s1):
    wid = lax.axis_index("s") * 2 + lax.axis_index("c")
    pltpu.sync_copy(idxp.at[wid], idx_v)
    xw = x.at[wid]
    ow = out.at[wid]
    bufs = (buf0, buf1)
    gsem = (g0, g1)
    ssem = (s0, s1)

    def start_gather(c):
        if c == _NFULL:
            return pltpu.async_copy(xw.at[idx_v.at[0, pl.ds(_NFULL * _CHUNK, 1)]],
                                    tail_v, gsem[c % 2])
        return pltpu.async_copy(xw.at[idx_v.at[0, pl.ds(c * _CHUNK, _CHUNK)]],
                                bufs[c % 2], gsem[c % 2])

    def start_store(c):
        if c == _NFULL:
            return pltpu.async_copy(tail_v, ow.at[pl.ds(_NFULL * _CHUNK, 1)],
                                    ssem[c % 2])
        return pltpu.async_copy(bufs[c % 2], ow.at[pl.ds(c * _CHUNK, _CHUNK)],
                                ssem[c % 2])

    nch = _NFULL + 1
    g_h = [None] * nch
    s_h = [None] * nch
    g_h[0] = start_gather(0)
    for c in range(nch):
        if c + 1 < nch:
            if c - 1 >= 0 and c + 1 < _NFULL:
                s_h[c - 1].wait()
            g_h[c + 1] = start_gather(c + 1)
        g_h[c].wait()
        s_h[c] = start_store(c)
    s_h[nch - 3].wait()
    s_h[nch - 2].wait()
    s_h[nch - 1].wait()


@functools.lru_cache(maxsize=1)
def _make_gather_call():
    return functools.partial(
        pl.kernel,
        out_type=jax.ShapeDtypeStruct((_B, _K, _C), jnp.float32),
        mesh=plsc.VectorSubcoreMesh(core_axis_name="c", subcore_axis_name="s"),
        scratch_types=[
            pltpu.VMEM((1, _PPAD), jnp.int32),
            pltpu.VMEM((_CHUNK, _C), jnp.float32),
            pltpu.VMEM((_CHUNK, _C), jnp.float32),
            pltpu.VMEM((1, _C), jnp.float32),
            pltpu.SemaphoreType.DMA,
            pltpu.SemaphoreType.DMA,
            pltpu.SemaphoreType.DMA,
            pltpu.SemaphoreType.DMA,
        ],
    )(_gather_body)


def kernel(x, layer_idx, requested_r):
    del layer_idx
    r_arr = jnp.asarray(requested_r, jnp.int32).reshape(1)
    idxp = _topk_call(r_arr, x)
    return _make_gather_call()(x, idxp)
